# baseline (speedup 1.0000x reference)
import functools

import jax
import jax.numpy as jnp
from jax import lax
from jax.experimental import pallas as pl
from jax.experimental.pallas import tpu as pltpu
from jax.experimental.pallas import tpu_sc as plsc

NC = 2
NS = 16
NW = NC * NS
LANE = 128
CH = 126 * LANE


@functools.lru_cache(maxsize=None)
def _build(S, D, B, V):
    CPW = D // NW
    vmain = (V // LANE) * LANE
    VP = vmain + LANE
    nch = vmain // CH
    assert nch * CH == vmain and nch % 2 == 0 and CPW == 2
    mesh = plsc.VectorSubcoreMesh(core_axis_name="c", subcore_axis_name="s")

    @functools.partial(
        pl.kernel,
        mesh=mesh,
        compiler_params=pltpu.CompilerParams(use_tc_tiling_on_sc=True),
        out_type=(
            jax.ShapeDtypeStruct((S, D, B), jnp.float32),
            jax.ShapeDtypeStruct((D * VP,), jnp.float32),
        ),
        scratch_types=[
            [pltpu.VMEM((CH,), jnp.float32)] * 2,
            [pltpu.VMEM((B,), jnp.int32)] * 4,
            [pltpu.VMEM((B,), jnp.float32)] * 4,
            [pltpu.SemaphoreType.DMA] * 2,
            [pltpu.SemaphoreType.DMA] * 4,
            [pltpu.SemaphoreType.DMA] * 4,
            [pltpu.SemaphoreType.DMA] * 4,
        ],
    )
    def emb(wt_hbm, wtail_hbm, idx_hbm, out_hbm, flat_hbm,
            ubuf, idx_v, gbuf, usems, gsems, isems, osems):
        wid = lax.axis_index("s") * NC + lax.axis_index("c")

        def uload(c, i, b):
            pltpu.async_copy(
                wt_hbm.at[c].at[pl.ds(i * CH, CH)], ubuf[b], usems[b]
            )

        def ustore(c, i, b):
            pltpu.make_async_copy(
                flat_hbm.at[pl.ds(0, CH)], ubuf[b], usems[b]
            ).wait()
            off = pl.multiple_of(c * VP + i * CH, LANE)
            pltpu.sync_copy(ubuf[b], flat_hbm.at[pl.ds(off, CH)])

        def utail(c):
            pltpu.sync_copy(wtail_hbm.at[c], ubuf[0].at[pl.ds(0, LANE)])
            off = pl.multiple_of(c * VP + vmain, LANE)
            pltpu.sync_copy(
                ubuf[0].at[pl.ds(0, LANE)],
                flat_hbm.at[pl.ds(off, LANE)],
            )

        def ifire(s, b):
            pltpu.async_copy(idx_hbm.at[s], idx_v[b], isems[b])

        def iwait(b):
            pltpu.make_async_copy(idx_hbm.at[0], idx_v[b], isems[b]).wait()

        def gfire(c, b):
            off = pl.multiple_of(c * VP, LANE)
            pltpu.async_copy(
                flat_hbm.at[pl.ds(off, VP)].at[idx_v[b]],
                gbuf[b], gsems[b],
            )

        def gwait(b):
            pltpu.make_async_copy(
                flat_hbm.at[pl.ds(0, B)], gbuf[b], gsems[b]
            ).wait()

        def ofire(s, c, b):
            pltpu.async_copy(gbuf[b], out_hbm.at[s].at[c], osems[b])

        def owait(b):
            pltpu.make_async_copy(
                flat_hbm.at[pl.ds(0, B)], gbuf[b], osems[b]
            ).wait()

        c0 = wid * CPW
        c1 = c0 + 1

        uload(c0, 0, 0)
        uload(c0, 1, 1)

        def p1(gi, carry):
            for b in range(2):
                i = 2 * gi + b
                ustore(c0, i, b)

                @pl.when(i + 2 < nch)
                def _():
                    uload(c0, i + 2, b)
            return carry

        lax.fori_loop(0, nch // 2, p1, 0)
        utail(c0)

        uload(c1, 0, 0)
        uload(c1, 1, 1)
        ifire(0, 0)

        def p2(gi, carry):
            for b in range(2):
                m = 2 * gi + b
                ustore(c1, m, b)

                @pl.when(m + 2 < nch)
                def _():
                    uload(c1, m + 2, b)

                @pl.when(m < S)
                def _():
                    iwait(b)

                @pl.when(jnp.logical_and(2 <= m, m < S))
                def _():
                    owait(b)

                @pl.when(m < S)
                def _():
                    gfire(c0, b)

                @pl.when(jnp.logical_and(1 <= m, m <= S))
                def _():
                    gwait(1 - b)
                    ofire(m - 1, c0, 1 - b)

                @pl.when(m + 1 < S)
                def _():
                    ifire(m + 1, 1 - b)
            return carry

        lax.fori_loop(0, nch // 2, p2, 0)
        utail(c1)

        NB = 4
        for r0 in range(NB):
            ifire(r0, r0)
        for r0 in range(NB - 1):
            if r0 < 2:
                owait(r0)
            iwait(r0)
            gfire(c1, r0)

        def p3(gi, carry):
            for b in range(NB):
                r = NB * gi + b
                b3 = (b + NB - 1) % NB

                @pl.when(jnp.logical_and(r + NB - 1 < S, r >= 1))
                def _():
                    owait(b3)
                    iwait(b3)
                    gfire(c1, b3)

                @pl.when(jnp.logical_and(r + NB - 1 < S, r < 1))
                def _():
                    iwait(b3)
                    gfire(c1, b3)

                @pl.when(r < S)
                def _():
                    gwait(b)
                    ofire(r, c1, b)

                @pl.when(r + NB < S)
                def _():
                    ifire(r + NB, b)
            return carry

        lax.fori_loop(0, (S + NB - 1) // NB, p3, 0)
        for k in range(NB):
            owait(k)

    return emb


def kernel(x, W):
    B, S = x.shape
    V, D = W.shape
    wt = W.T
    wtail = jnp.pad(W[(V // LANE) * LANE:], ((0, LANE - V % LANE), (0, 0))).T
    idxt = x.T
    out3, _ = _build(S, D, B, V)(wt, wtail, idxt)
    return out3.transpose(2, 0, 1)
